# Initial kernel scaffold; baseline (speedup 1.0000x reference)
import functools

import jax
import jax.numpy as jnp
import numpy as np
from jax import lax
from jax.experimental import pallas as pl
from jax.experimental.pallas import tpu as pltpu
from jax.experimental.pallas import tpu_sc as plsc

N = 10000
E = 320000
EMBED = 16
HID = 128
HEADS = 8
DH = 16
VOCAB = 20000
L = 2
NEG = -1e30

_MESH = plsc.VectorSubcoreMesh(core_axis_name="c", subcore_axis_name="s")
NW = 32
ROWS_PER_TILE = N // 16


def _wid():
    return lax.axis_index("c") * 16 + lax.axis_index("s")


def _sc_embed_gather(table, idx2d, n_chunks):
    @functools.partial(
        pl.kernel,
        out_type=jax.ShapeDtypeStruct((n_chunks * 1024, 16), jnp.float32),
        mesh=_MESH,
        scratch_types=[
            pltpu.VMEM((8, 128), jnp.int32),
            pltpu.VMEM((1024, 16), jnp.float32),
            pltpu.SemaphoreType.DMA,
        ],
    )
    def k(table_h, idx_h, out_h, idx_v, rows_v, sem):
        w = _wid()
        n_iter = (n_chunks + NW - 1) // NW

        def body(i, carry):
            t = w + i * NW

            @pl.when(t < n_chunks)
            def _():
                pltpu.sync_copy(idx_h.at[pl.ds(t * 8, 8)], idx_v)
                descs = [
                    pltpu.async_copy(
                        table_h.at[idx_v.at[j]],
                        rows_v.at[pl.ds(j * 128, 128)],
                        sem,
                    )
                    for j in range(8)
                ]
                for d in descs:
                    d.wait()
                pltpu.sync_copy(rows_v, out_h.at[pl.ds(t * 1024, 1024)])

            return carry

        lax.fori_loop(0, n_iter, body, 0)

    return k(table, idx2d)


def _sc_edge_gather2(q, k, dst2d, src2d):
    nch = E // 256

    @functools.partial(
        pl.kernel,
        out_type=(
            jax.ShapeDtypeStruct((E, HID), jnp.float32),
            jax.ShapeDtypeStruct((E, HID), jnp.float32),
        ),
        mesh=_MESH,
        scratch_types=[
            pltpu.VMEM((2, 128), jnp.int32),
            pltpu.VMEM((2, 128), jnp.int32),
            pltpu.VMEM((256, HID), jnp.float32),
            pltpu.VMEM((256, HID), jnp.float32),
            pltpu.SemaphoreType.DMA,
        ],
    )
    def k_(q_h, k_h, dst_h, src_h, qd_h, ks_h, dbuf, sbuf, qv, kv, sem):
        w = _wid()
        n_iter = (nch + NW - 1) // NW

        def body(i, carry):
            t = w + i * NW

            @pl.when(t < nch)
            def _():
                pltpu.sync_copy(dst_h.at[pl.ds(t * 2, 2)], dbuf)
                pltpu.sync_copy(src_h.at[pl.ds(t * 2, 2)], sbuf)
                descs = []
                for j in range(2):
                    descs.append(pltpu.async_copy(
                        q_h.at[dbuf.at[j]], qv.at[pl.ds(j * 128, 128)], sem))
                    descs.append(pltpu.async_copy(
                        k_h.at[sbuf.at[j]], kv.at[pl.ds(j * 128, 128)], sem))
                for d in descs:
                    d.wait()
                pltpu.sync_copy(qv, qd_h.at[pl.ds(t * 256, 256)])
                pltpu.sync_copy(kv, ks_h.at[pl.ds(t * 256, 256)])

            return carry

        lax.fori_loop(0, n_iter, body, 0)

    return k_(q, k, dst2d, src2d)


def _sc_den_scatter(logits, gmax16, dst2d, zeros16):
    nch = E // 512

    @functools.partial(
        pl.kernel,
        out_type=jax.ShapeDtypeStruct((2, N, 16), jnp.float32),
        mesh=_MESH,
        scratch_types=[
            pltpu.VMEM((512, 16), jnp.float32),
            pltpu.VMEM((512, 16), jnp.float32),
            pltpu.VMEM((4, 128), jnp.int32),
            pltpu.VMEM((16,), jnp.float32),
            pltpu.VMEM_SHARED((N, 16), jnp.float32),
            pltpu.SemaphoreType.DMA,
        ],
    )
    def k_(lg_h, g_h, dst_h, z_h, den_h, lg, ex, dbuf, gv, den_sp, sem):
        cid = lax.axis_index("c")
        sid = lax.axis_index("s")
        w = cid * 16 + sid
        pltpu.sync_copy(z_h.at[pl.ds(sid * ROWS_PER_TILE, ROWS_PER_TILE)],
                        den_sp.at[pl.ds(sid * ROWS_PER_TILE, ROWS_PER_TILE)])
        pltpu.sync_copy(g_h, gv)
        plsc.subcore_barrier()
        gvec = gv[...]
        n_iter = (nch + NW - 1) // NW

        def body(i, carry):
            t = w + i * NW

            @pl.when(t < nch)
            def _():
                pltpu.sync_copy(lg_h.at[pl.ds(t * 512, 512)], lg)
                pltpu.sync_copy(dst_h.at[pl.ds(t * 4, 4)], dbuf)

                def inner(b, c2):
                    ex[b] = jnp.exp(lg[b] - gvec)
                    return c2

                lax.fori_loop(0, 512, inner, 0)
                for j in range(4):
                    pltpu.sync_copy(ex.at[pl.ds(j * 128, 128)],
                                    den_sp.at[dbuf.at[j]], add=True)

            return carry

        lax.fori_loop(0, n_iter, body, 0)
        plsc.subcore_barrier()
        pltpu.sync_copy(den_sp.at[pl.ds(sid * ROWS_PER_TILE, ROWS_PER_TILE)],
                        den_h.at[cid].at[pl.ds(sid * ROWS_PER_TILE,
                                               ROWS_PER_TILE)])

    return k_(logits, gmax16, dst2d, zeros16)


def _sc_agg_scatter(logits, gmax16, e_l, v, den0, den1, dst2d, src2d,
                    zeros128):
    nch = E // 256

    @functools.partial(
        pl.kernel,
        out_type=jax.ShapeDtypeStruct((2, N, HID), jnp.float32),
        mesh=_MESH,
        scratch_types=[
            pltpu.VMEM((256, 16), jnp.float32),
            pltpu.VMEM((256, HID), jnp.float32),
            pltpu.VMEM((256, HID), jnp.float32),
            pltpu.VMEM((256, 16), jnp.float32),
            pltpu.VMEM((256, 16), jnp.float32),
            pltpu.VMEM((256, 16), jnp.float32),
            pltpu.VMEM((256, HID), jnp.float32),
            pltpu.VMEM((2, 128), jnp.int32),
            pltpu.VMEM((2, 128), jnp.int32),
            pltpu.VMEM((16,), jnp.float32),
            pltpu.VMEM_SHARED((N, HID), jnp.float32),
            pltpu.SemaphoreType.DMA,
        ],
    )
    def k_(lg_h, g_h, e_h, v_h, d0_h, d1_h, dst_h, src_h, z_h, agg_h,
           lg, ev, vs, d0, d1, al, av, dbuf, sbuf, gv, agg_sp, sem):
        cid = lax.axis_index("c")
        sid = lax.axis_index("s")
        w = cid * 16 + sid
        pltpu.sync_copy(z_h.at[pl.ds(sid * ROWS_PER_TILE, ROWS_PER_TILE)],
                        agg_sp.at[pl.ds(sid * ROWS_PER_TILE, ROWS_PER_TILE)])
        pltpu.sync_copy(g_h, gv)
        plsc.subcore_barrier()
        gvec = gv[...]
        n_iter = (nch + NW - 1) // NW

        def body(i, carry):
            t = w + i * NW

            @pl.when(t < nch)
            def _():
                pltpu.sync_copy(dst_h.at[pl.ds(t * 2, 2)], dbuf)
                pltpu.sync_copy(src_h.at[pl.ds(t * 2, 2)], sbuf)
                pltpu.sync_copy(lg_h.at[pl.ds(t * 256, 256)], lg)
                pltpu.sync_copy(e_h.at[pl.ds(t * 256, 256)], ev)
                descs = []
                for j in range(2):
                    descs.append(pltpu.async_copy(
                        v_h.at[sbuf.at[j]], vs.at[pl.ds(j * 128, 128)], sem))
                    descs.append(pltpu.async_copy(
                        d0_h.at[dbuf.at[j]], d0.at[pl.ds(j * 128, 128)], sem))
                    descs.append(pltpu.async_copy(
                        d1_h.at[dbuf.at[j]], d1.at[pl.ds(j * 128, 128)], sem))
                for d in descs:
                    d.wait()

                def inner(b, c2):
                    exv = jnp.exp(lg[b] - gvec)
                    denv = d0[b] + d1[b] + 1e-16
                    al[b] = exv / denv
                    return c2

                lax.fori_loop(0, 256, inner, 0)

                def inner2(b, c2):
                    for h in range(HEADS):
                        s = al[b, h]
                        av[b, pl.ds(h * 16, 16)] = (
                            vs[b, pl.ds(h * 16, 16)] + ev[b, pl.ds(h * 16, 16)]
                        ) * s
                    return c2

                lax.fori_loop(0, 256, inner2, 0)
                for j in range(2):
                    pltpu.sync_copy(av.at[pl.ds(j * 128, 128)],
                                    agg_sp.at[dbuf.at[j]], add=True)

            return carry

        lax.fori_loop(0, n_iter, body, 0)
        plsc.subcore_barrier()
        pltpu.sync_copy(agg_sp.at[pl.ds(sid * ROWS_PER_TILE, ROWS_PER_TILE)],
                        agg_h.at[cid].at[pl.ds(sid * ROWS_PER_TILE,
                                               ROWS_PER_TILE)])

    return k_(logits, gmax16, e_l, v, den0, den1, dst2d, src2d, zeros128)


def _tc_matmul_bias(x, w, b, blk):
    m, kdim = x.shape
    n = w.shape[1]
    grid = (m + blk - 1) // blk

    def body(x_r, w_r, b_r, o_r):
        o_r[...] = jnp.dot(x_r[...], w_r[...],
                           preferred_element_type=jnp.float32) + b_r[...]

    return pl.pallas_call(
        body,
        grid=(grid,),
        in_specs=[
            pl.BlockSpec((blk, kdim), lambda i: (i, 0)),
            pl.BlockSpec((kdim, n), lambda i: (0, 0)),
            pl.BlockSpec((1, n), lambda i: (0, 0)),
        ],
        out_specs=pl.BlockSpec((blk, n), lambda i: (i, 0)),
        out_shape=jax.ShapeDtypeStruct((m, n), jnp.float32),
    )(x, w, b)


def _tc_proj4(x, wq, wk, wv, wskip, bskip):
    blk = 1024
    grid = (N + blk - 1) // blk

    def body(x_r, wq_r, wk_r, wv_r, ws_r, bs_r, q_r, k_r, v_r, xr_r):
        xb = x_r[...]
        q_r[...] = jnp.dot(xb, wq_r[...], preferred_element_type=jnp.float32)
        k_r[...] = jnp.dot(xb, wk_r[...], preferred_element_type=jnp.float32)
        v_r[...] = jnp.dot(xb, wv_r[...], preferred_element_type=jnp.float32)
        xr_r[...] = jnp.dot(xb, ws_r[...],
                            preferred_element_type=jnp.float32) + bs_r[...]

    o = jax.ShapeDtypeStruct((N, HID), jnp.float32)
    wspec = pl.BlockSpec((HID, HID), lambda i: (0, 0))
    return pl.pallas_call(
        body,
        grid=(grid,),
        in_specs=[pl.BlockSpec((blk, HID), lambda i: (i, 0)),
                  wspec, wspec, wspec, wspec,
                  pl.BlockSpec((1, HID), lambda i: (0, 0))],
        out_specs=[pl.BlockSpec((blk, HID), lambda i: (i, 0))] * 4,
        out_shape=[o, o, o, o],
    )(x, wq, wk, wv, wskip, bskip)


def _tc_logits(qd, ks, ef, we, sel):
    blk = 512
    grid = E // blk

    def body(qd_r, ks_r, ef_r, we_r, sel_r, lg_r, e_r, g_r, mx_r):
        i = pl.program_id(0)
        e = jnp.dot(ef_r[...], we_r[...], preferred_element_type=jnp.float32)
        e_r[...] = e
        prod = qd_r[...] * (ks_r[...] + e) * (1.0 / np.sqrt(DH))
        l8 = jnp.dot(prod, sel_r[...], preferred_element_type=jnp.float32)
        lg_r[...] = jnp.concatenate(
            [l8, jnp.full((blk, 8), NEG, jnp.float32)], axis=1)
        bm = jnp.max(l8, axis=0)[None, :]
        prev = jnp.where(i == 0, jnp.full((1, 8), NEG, jnp.float32), mx_r[...])
        cur = jnp.maximum(prev, bm)
        mx_r[...] = cur
        g_r[...] = jnp.concatenate([cur, jnp.zeros((1, 8), jnp.float32)],
                                   axis=1)

    return pl.pallas_call(
        body,
        grid=(grid,),
        in_specs=[
            pl.BlockSpec((blk, HID), lambda i: (i, 0)),
            pl.BlockSpec((blk, HID), lambda i: (i, 0)),
            pl.BlockSpec((blk, HID), lambda i: (i, 0)),
            pl.BlockSpec((HID, HID), lambda i: (0, 0)),
            pl.BlockSpec((HID, 8), lambda i: (0, 0)),
        ],
        out_specs=[
            pl.BlockSpec((blk, 16), lambda i: (i, 0)),
            pl.BlockSpec((blk, HID), lambda i: (i, 0)),
            pl.BlockSpec((1, 16), lambda i: (0, 0)),
        ],
        out_shape=[
            jax.ShapeDtypeStruct((E, 16), jnp.float32),
            jax.ShapeDtypeStruct((E, HID), jnp.float32),
            jax.ShapeDtypeStruct((1, 16), jnp.float32),
        ],
        scratch_shapes=[pltpu.VMEM((1, 8), jnp.float32)],
    )(qd, ks, ef, we, sel)


def _tc_post(x, agg0, agg1, xr, wba, wbx, g1, b1, wf1, bf1, wf2, bf2, g2, b2):
    blk = 1024
    grid = (N + blk - 1) // blk

    def ln(y, g, b):
        m = jnp.mean(y, axis=-1, keepdims=True)
        v = jnp.mean((y - m) ** 2, axis=-1, keepdims=True)
        return g * (y - m) / jnp.sqrt(v + 1e-5) + b

    def body(x_r, a0_r, a1_r, xr_r, wba_r, wbx_r, g1_r, b1_r,
             wf1_r, bf1_r, wf2_r, bf2_r, g2_r, b2_r, o_r):
        agg = a0_r[...] + a1_r[...]
        xrb = xr_r[...]
        bl = (jnp.dot(agg, wba_r[...], preferred_element_type=jnp.float32)
              + jnp.dot(xrb, wbx_r[...], preferred_element_type=jnp.float32))
        beta = jax.nn.sigmoid(bl)
        h = beta * xrb + (1.0 - beta) * agg
        y = ln(x_r[...] + h, g1_r[...], b1_r[...])
        h2 = jnp.dot(
            jax.nn.gelu(jnp.dot(y, wf1_r[...],
                                preferred_element_type=jnp.float32)
                        + bf1_r[...]),
            wf2_r[...], preferred_element_type=jnp.float32) + bf2_r[...]
        o_r[...] = ln(y + h2, g2_r[...], b2_r[...])

    nblk = pl.BlockSpec((blk, HID), lambda i: (i, 0))
    row = pl.BlockSpec((1, HID), lambda i: (0, 0))
    return pl.pallas_call(
        body,
        grid=(grid,),
        in_specs=[
            nblk, nblk, nblk, nblk,
            pl.BlockSpec((HID, 1), lambda i: (0, 0)),
            pl.BlockSpec((HID, 1), lambda i: (0, 0)),
            row, row,
            pl.BlockSpec((HID, 4 * HID), lambda i: (0, 0)),
            pl.BlockSpec((1, 4 * HID), lambda i: (0, 0)),
            pl.BlockSpec((4 * HID, HID), lambda i: (0, 0)),
            row, row, row,
        ],
        out_specs=nblk,
        out_shape=jax.ShapeDtypeStruct((N, HID), jnp.float32),
    )(x, agg0, agg1, xr, wba, wbx, g1, b1, wf1, bf1, wf2, bf2, g2, b2)


def kernel(x_cont, node_cat, lookahead_cat, package_postal, edge_index,
           edge_cont, edge_cat, node_tables, lookahead_tables, edge_tables,
           postal_table, W_node, b_node, W_edge, b_edge, Wq, Wk, Wv, We,
           Wskip, bskip, Wbeta, ln1_g, ln1_b, Wf1, bf1, Wf2, bf2,
           ln2_g, ln2_b):
    i32 = jnp.int32
    f32 = jnp.float32

    stacked = jnp.concatenate([
        node_tables.reshape(-1, EMBED),
        lookahead_tables.reshape(-1, EMBED),
        postal_table,
        edge_tables.reshape(-1, EMBED),
    ], axis=0)
    offn = (jnp.arange(9, dtype=i32) * VOCAB)[None, :]
    offl = ((9 + jnp.arange(7, dtype=i32)) * VOCAB)[None, :]
    offe = ((17 + jnp.arange(9, dtype=i32)) * VOCAB)[None, :]
    idx_n = jnp.concatenate([
        node_cat.astype(i32) + offn,
        lookahead_cat.astype(i32) + offl,
        package_postal.astype(i32) + 16 * VOCAB,
    ], axis=1).reshape(-1)
    idx_e = (edge_cat.astype(i32) + offe).reshape(-1)
    idx_all = jnp.concatenate([idx_n, idx_e])
    total = idx_all.shape[0]
    n_chunks = (total + 1023) // 1024
    pad = n_chunks * 1024 - total
    idx_all = jnp.concatenate([idx_all, jnp.zeros((pad,), i32)])
    idx2d = idx_all.reshape(n_chunks * 8, 128)

    gathered = _sc_embed_gather(stacked, idx2d, n_chunks)
    node_emb = gathered[:180000].reshape(N, 18 * EMBED)
    edge_emb = gathered[180000:180000 + 9 * E].reshape(E, 9 * EMBED)

    x_in = jnp.concatenate([x_cont, node_emb], axis=1)
    x = _tc_matmul_bias(x_in, W_node, b_node[None, :], 1024)
    ef_in = jnp.concatenate([edge_cont, edge_emb], axis=1)
    e_feat = _tc_matmul_bias(ef_in, W_edge, b_edge[None, :], 2048)

    src2d = edge_index[0].astype(i32).reshape(E // 128, 128)
    dst2d = edge_index[1].astype(i32).reshape(E // 128, 128)

    sel = jnp.array(np.repeat(np.eye(8, dtype=np.float32), DH, axis=0))
    z16 = jnp.zeros((N, 16), f32)
    z128 = jnp.zeros((N, HID), f32)

    for l in range(L):
        q, k, v, xr = _tc_proj4(x, Wq[l], Wk[l], Wv[l], Wskip[l],
                                bskip[l][None, :])
        qd, ks = _sc_edge_gather2(q, k, dst2d, src2d)
        logits, e_l, g16 = _tc_logits(qd, ks, e_feat, We[l], sel)
        g16 = g16.reshape(16)
        den = _sc_den_scatter(logits, g16, dst2d, z16)
        agg = _sc_agg_scatter(logits, g16, e_l, v, den[0], den[1],
                              dst2d, src2d, z128)
        wb = Wbeta[l]
        wba = wb[:HID] + wb[2 * HID:]
        wbx = wb[HID:2 * HID] - wb[2 * HID:]
        x = _tc_post(x, agg[0], agg[1], xr, wba, wbx,
                     ln1_g[l][None, :], ln1_b[l][None, :],
                     Wf1[l], bf1[l][None, :], Wf2[l], bf2[l][None, :],
                     ln2_g[l][None, :], ln2_b[l][None, :])
    return x


# speedup vs baseline: 10.9601x; 10.9601x over previous
import functools

import jax
import jax.numpy as jnp
import numpy as np
from jax import lax
from jax.experimental import pallas as pl
from jax.experimental.pallas import tpu as pltpu
from jax.experimental.pallas import tpu_sc as plsc

N = 10000
E = 320000
EMBED = 16
HID = 128
HEADS = 8
DH = 16
VOCAB = 20000
L = 2
NEG = -1e30

_MESH = plsc.VectorSubcoreMesh(core_axis_name="c", subcore_axis_name="s")
NW = 32
ROWS_PER_TILE = N // 16


def _wid():
    return lax.axis_index("c") * 16 + lax.axis_index("s")


def _sc_embed_gather(table, idx2d, n_chunks):
    @functools.partial(
        pl.kernel,
        out_type=jax.ShapeDtypeStruct((n_chunks * 1024, 16), jnp.float32),
        mesh=_MESH,
        compiler_params=pltpu.CompilerParams(use_tc_tiling_on_sc=False),
        scratch_types=[
            pltpu.VMEM((8, 128), jnp.int32),
            pltpu.VMEM((1024, 16), jnp.float32),
            pltpu.SemaphoreType.DMA,
        ],
    )
    def k(table_h, idx_h, out_h, idx_v, rows_v, sem):
        w = _wid()
        n_iter = (n_chunks + NW - 1) // NW

        def body(i, carry):
            t = w + i * NW

            @pl.when(t < n_chunks)
            def _():
                pltpu.sync_copy(idx_h.at[pl.ds(t * 8, 8)], idx_v)
                descs = [
                    pltpu.async_copy(
                        table_h.at[idx_v.at[j]],
                        rows_v.at[pl.ds(j * 128, 128)],
                        sem,
                    )
                    for j in range(8)
                ]
                for d in descs:
                    d.wait()
                pltpu.sync_copy(rows_v, out_h.at[pl.ds(t * 1024, 1024)])

            return carry

        lax.fori_loop(0, n_iter, body, 0)

    return k(table, idx2d)


def _sc_edge_gather2(q, k, dst2d, src2d):
    nch = E // 256

    @functools.partial(
        pl.kernel,
        out_type=(
            jax.ShapeDtypeStruct((E, HID), jnp.float32),
            jax.ShapeDtypeStruct((E, HID), jnp.float32),
        ),
        mesh=_MESH,
        compiler_params=pltpu.CompilerParams(use_tc_tiling_on_sc=False),
        scratch_types=[
            pltpu.VMEM((2, 128), jnp.int32),
            pltpu.VMEM((2, 128), jnp.int32),
            pltpu.VMEM((256, HID), jnp.float32),
            pltpu.VMEM((256, HID), jnp.float32),
            pltpu.SemaphoreType.DMA,
        ],
    )
    def k_(q_h, k_h, dst_h, src_h, qd_h, ks_h, dbuf, sbuf, qv, kv, sem):
        w = _wid()
        n_iter = (nch + NW - 1) // NW

        def body(i, carry):
            t = w + i * NW

            @pl.when(t < nch)
            def _():
                pltpu.sync_copy(dst_h.at[pl.ds(t * 2, 2)], dbuf)
                pltpu.sync_copy(src_h.at[pl.ds(t * 2, 2)], sbuf)
                descs = []
                for j in range(2):
                    descs.append(pltpu.async_copy(
                        q_h.at[dbuf.at[j]], qv.at[pl.ds(j * 128, 128)], sem))
                    descs.append(pltpu.async_copy(
                        k_h.at[sbuf.at[j]], kv.at[pl.ds(j * 128, 128)], sem))
                for d in descs:
                    d.wait()
                pltpu.sync_copy(qv, qd_h.at[pl.ds(t * 256, 256)])
                pltpu.sync_copy(kv, ks_h.at[pl.ds(t * 256, 256)])

            return carry

        lax.fori_loop(0, n_iter, body, 0)

    return k_(q, k, dst2d, src2d)


def _sc_den_scatter(logits, gmax16, dst2d, zeros16):
    nch = E // 512

    @functools.partial(
        pl.kernel,
        out_type=jax.ShapeDtypeStruct((2, N, 16), jnp.float32),
        mesh=_MESH,
        compiler_params=pltpu.CompilerParams(use_tc_tiling_on_sc=False),
        scratch_types=[
            pltpu.VMEM((512, 16), jnp.float32),
            pltpu.VMEM((512, 16), jnp.float32),
            pltpu.VMEM((4, 128), jnp.int32),
            pltpu.VMEM((16,), jnp.float32),
            pltpu.VMEM_SHARED((N, 16), jnp.float32),
            pltpu.SemaphoreType.DMA,
        ],
    )
    def k_(lg_h, g_h, dst_h, z_h, den_h, lg, ex, dbuf, gv, den_sp, sem):
        cid = lax.axis_index("c")
        sid = lax.axis_index("s")
        w = cid * 16 + sid
        pltpu.sync_copy(z_h.at[pl.ds(sid * ROWS_PER_TILE, ROWS_PER_TILE)],
                        den_sp.at[pl.ds(sid * ROWS_PER_TILE, ROWS_PER_TILE)])
        pltpu.sync_copy(g_h, gv)
        plsc.subcore_barrier()
        gvec = gv[...]
        n_iter = (nch + NW - 1) // NW

        def body(i, carry):
            t = w + i * NW

            @pl.when(t < nch)
            def _():
                pltpu.sync_copy(lg_h.at[pl.ds(t * 512, 512)], lg)
                pltpu.sync_copy(dst_h.at[pl.ds(t * 4, 4)], dbuf)

                def inner(b, c2):
                    ex[b] = jnp.exp(lg[b] - gvec)
                    return c2

                lax.fori_loop(0, 512, inner, 0)
                for j in range(4):
                    pltpu.sync_copy(ex.at[pl.ds(j * 128, 128)],
                                    den_sp.at[dbuf.at[j]], add=True)

            return carry

        lax.fori_loop(0, n_iter, body, 0)
        plsc.subcore_barrier()
        pltpu.sync_copy(den_sp.at[pl.ds(sid * ROWS_PER_TILE, ROWS_PER_TILE)],
                        den_h.at[cid].at[pl.ds(sid * ROWS_PER_TILE,
                                               ROWS_PER_TILE)])

    return k_(logits, gmax16, dst2d, zeros16)


def _sc_agg_scatter(logits, gmax16, e_l, v, den0, den1, dst2d, src2d,
                    zeros128):
    nch = E // 128

    @functools.partial(
        pl.kernel,
        out_type=jax.ShapeDtypeStruct((2, N, HID), jnp.float32),
        mesh=_MESH,
        compiler_params=pltpu.CompilerParams(use_tc_tiling_on_sc=False),
        scratch_types=[
            pltpu.VMEM((128, 16), jnp.float32),
            pltpu.VMEM((128, HID), jnp.float32),
            pltpu.VMEM((128, HID), jnp.float32),
            pltpu.VMEM((128, 16), jnp.float32),
            pltpu.VMEM((128, 16), jnp.float32),
            pltpu.VMEM((1, 128), jnp.int32),
            pltpu.VMEM((1, 128), jnp.int32),
            pltpu.VMEM((16,), jnp.float32),
            pltpu.VMEM_SHARED((N, HID), jnp.float32),
            pltpu.SemaphoreType.DMA,
        ],
    )
    def k_(lg_h, g_h, e_h, v_h, d0_h, d1_h, dst_h, src_h, z_h, agg_h,
           lg, ev, vs, d0, d1, dbuf, sbuf, gv, agg_sp, sem):
        cid = lax.axis_index("c")
        sid = lax.axis_index("s")
        w = cid * 16 + sid
        pltpu.sync_copy(z_h.at[pl.ds(sid * ROWS_PER_TILE, ROWS_PER_TILE)],
                        agg_sp.at[pl.ds(sid * ROWS_PER_TILE, ROWS_PER_TILE)])
        pltpu.sync_copy(g_h, gv)
        plsc.subcore_barrier()
        gvec = gv[...]
        n_iter = (nch + NW - 1) // NW

        def body(i, carry):
            t = w + i * NW

            @pl.when(t < nch)
            def _():
                pltpu.sync_copy(dst_h.at[pl.ds(t, 1)], dbuf)
                pltpu.sync_copy(src_h.at[pl.ds(t, 1)], sbuf)
                pltpu.sync_copy(lg_h.at[pl.ds(t * 128, 128)], lg)
                pltpu.sync_copy(e_h.at[pl.ds(t * 128, 128)], ev)
                descs = [
                    pltpu.async_copy(v_h.at[sbuf.at[0]], vs, sem),
                    pltpu.async_copy(d0_h.at[dbuf.at[0]], d0, sem),
                    pltpu.async_copy(d1_h.at[dbuf.at[0]], d1, sem),
                ]
                for d in descs:
                    d.wait()

                def inner(b, c2):
                    exv = jnp.exp(lg[b] - gvec)
                    denv = d0[b] + d1[b] + 1e-16
                    alv = exv / denv
                    for h in range(HEADS):
                        ev[b, pl.ds(h * 16, 16)] = (
                            vs[b, pl.ds(h * 16, 16)] + ev[b, pl.ds(h * 16, 16)]
                        ) * alv[h]
                    return c2

                lax.fori_loop(0, 128, inner, 0)
                pltpu.sync_copy(ev, agg_sp.at[dbuf.at[0]], add=True)

            return carry

        lax.fori_loop(0, n_iter, body, 0)
        plsc.subcore_barrier()
        pltpu.sync_copy(agg_sp.at[pl.ds(sid * ROWS_PER_TILE, ROWS_PER_TILE)],
                        agg_h.at[cid].at[pl.ds(sid * ROWS_PER_TILE,
                                               ROWS_PER_TILE)])

    return k_(logits, gmax16, e_l, v, den0, den1, dst2d, src2d, zeros128)


def _tc_matmul_bias(x, w, b, blk):
    m, kdim = x.shape
    n = w.shape[1]
    grid = (m + blk - 1) // blk

    def body(x_r, w_r, b_r, o_r):
        o_r[...] = jnp.dot(x_r[...], w_r[...],
                           preferred_element_type=jnp.float32) + b_r[...]

    return pl.pallas_call(
        body,
        grid=(grid,),
        in_specs=[
            pl.BlockSpec((blk, kdim), lambda i: (i, 0)),
            pl.BlockSpec((kdim, n), lambda i: (0, 0)),
            pl.BlockSpec((1, n), lambda i: (0, 0)),
        ],
        out_specs=pl.BlockSpec((blk, n), lambda i: (i, 0)),
        out_shape=jax.ShapeDtypeStruct((m, n), jnp.float32),
    )(x, w, b)


def _tc_proj4(x, wq, wk, wv, wskip, bskip):
    blk = 1024
    grid = (N + blk - 1) // blk

    def body(x_r, wq_r, wk_r, wv_r, ws_r, bs_r, q_r, k_r, v_r, xr_r):
        xb = x_r[...]
        q_r[...] = jnp.dot(xb, wq_r[...], preferred_element_type=jnp.float32)
        k_r[...] = jnp.dot(xb, wk_r[...], preferred_element_type=jnp.float32)
        v_r[...] = jnp.dot(xb, wv_r[...], preferred_element_type=jnp.float32)
        xr_r[...] = jnp.dot(xb, ws_r[...],
                            preferred_element_type=jnp.float32) + bs_r[...]

    o = jax.ShapeDtypeStruct((N, HID), jnp.float32)
    wspec = pl.BlockSpec((HID, HID), lambda i: (0, 0))
    return pl.pallas_call(
        body,
        grid=(grid,),
        in_specs=[pl.BlockSpec((blk, HID), lambda i: (i, 0)),
                  wspec, wspec, wspec, wspec,
                  pl.BlockSpec((1, HID), lambda i: (0, 0))],
        out_specs=[pl.BlockSpec((blk, HID), lambda i: (i, 0))] * 4,
        out_shape=[o, o, o, o],
    )(x, wq, wk, wv, wskip, bskip)


def _tc_logits(qd, ks, ef, we, sel):
    blk = 512
    grid = E // blk

    def body(qd_r, ks_r, ef_r, we_r, sel_r, lg_r, e_r, g_r, mx_r):
        i = pl.program_id(0)
        e = jnp.dot(ef_r[...], we_r[...], preferred_element_type=jnp.float32)
        e_r[...] = e
        prod = qd_r[...] * (ks_r[...] + e) * (1.0 / np.sqrt(DH))
        l8 = jnp.dot(prod, sel_r[...], preferred_element_type=jnp.float32)
        lg_r[...] = jnp.concatenate(
            [l8, jnp.full((blk, 8), NEG, jnp.float32)], axis=1)
        bm = jnp.max(l8, axis=0)[None, :]
        prev = jnp.where(i == 0, jnp.full((1, 8), NEG, jnp.float32), mx_r[...])
        cur = jnp.maximum(prev, bm)
        mx_r[...] = cur
        g_r[...] = jnp.concatenate([cur, jnp.zeros((1, 8), jnp.float32)],
                                   axis=1)

    return pl.pallas_call(
        body,
        grid=(grid,),
        in_specs=[
            pl.BlockSpec((blk, HID), lambda i: (i, 0)),
            pl.BlockSpec((blk, HID), lambda i: (i, 0)),
            pl.BlockSpec((blk, HID), lambda i: (i, 0)),
            pl.BlockSpec((HID, HID), lambda i: (0, 0)),
            pl.BlockSpec((HID, 8), lambda i: (0, 0)),
        ],
        out_specs=[
            pl.BlockSpec((blk, 16), lambda i: (i, 0)),
            pl.BlockSpec((blk, HID), lambda i: (i, 0)),
            pl.BlockSpec((1, 16), lambda i: (0, 0)),
        ],
        out_shape=[
            jax.ShapeDtypeStruct((E, 16), jnp.float32),
            jax.ShapeDtypeStruct((E, HID), jnp.float32),
            jax.ShapeDtypeStruct((1, 16), jnp.float32),
        ],
        scratch_shapes=[pltpu.VMEM((1, 8), jnp.float32)],
    )(qd, ks, ef, we, sel)


def _tc_post(x, agg0, agg1, xr, wba, wbx, g1, b1, wf1, bf1, wf2, bf2, g2, b2):
    blk = 1024
    grid = (N + blk - 1) // blk

    def ln(y, g, b):
        m = jnp.mean(y, axis=-1, keepdims=True)
        v = jnp.mean((y - m) ** 2, axis=-1, keepdims=True)
        return g * (y - m) / jnp.sqrt(v + 1e-5) + b

    def body(x_r, a0_r, a1_r, xr_r, wba_r, wbx_r, g1_r, b1_r,
             wf1_r, bf1_r, wf2_r, bf2_r, g2_r, b2_r, o_r):
        agg = a0_r[...] + a1_r[...]
        xrb = xr_r[...]
        bl = (jnp.dot(agg, wba_r[...], preferred_element_type=jnp.float32)
              + jnp.dot(xrb, wbx_r[...], preferred_element_type=jnp.float32))
        beta = jax.nn.sigmoid(bl)
        h = beta * xrb + (1.0 - beta) * agg
        y = ln(x_r[...] + h, g1_r[...], b1_r[...])
        h2 = jnp.dot(
            jax.nn.gelu(jnp.dot(y, wf1_r[...],
                                preferred_element_type=jnp.float32)
                        + bf1_r[...]),
            wf2_r[...], preferred_element_type=jnp.float32) + bf2_r[...]
        o_r[...] = ln(y + h2, g2_r[...], b2_r[...])

    nblk = pl.BlockSpec((blk, HID), lambda i: (i, 0))
    row = pl.BlockSpec((1, HID), lambda i: (0, 0))
    return pl.pallas_call(
        body,
        grid=(grid,),
        in_specs=[
            nblk, nblk, nblk, nblk,
            pl.BlockSpec((HID, 1), lambda i: (0, 0)),
            pl.BlockSpec((HID, 1), lambda i: (0, 0)),
            row, row,
            pl.BlockSpec((HID, 4 * HID), lambda i: (0, 0)),
            pl.BlockSpec((1, 4 * HID), lambda i: (0, 0)),
            pl.BlockSpec((4 * HID, HID), lambda i: (0, 0)),
            row, row, row,
        ],
        out_specs=nblk,
        out_shape=jax.ShapeDtypeStruct((N, HID), jnp.float32),
    )(x, agg0, agg1, xr, wba, wbx, g1, b1, wf1, bf1, wf2, bf2, g2, b2)


def kernel(x_cont, node_cat, lookahead_cat, package_postal, edge_index,
           edge_cont, edge_cat, node_tables, lookahead_tables, edge_tables,
           postal_table, W_node, b_node, W_edge, b_edge, Wq, Wk, Wv, We,
           Wskip, bskip, Wbeta, ln1_g, ln1_b, Wf1, bf1, Wf2, bf2,
           ln2_g, ln2_b):
    i32 = jnp.int32
    f32 = jnp.float32

    stacked = jnp.concatenate([
        node_tables.reshape(-1, EMBED),
        lookahead_tables.reshape(-1, EMBED),
        postal_table,
        edge_tables.reshape(-1, EMBED),
    ], axis=0)
    offn = (jnp.arange(9, dtype=i32) * VOCAB)[None, :]
    offl = ((9 + jnp.arange(7, dtype=i32)) * VOCAB)[None, :]
    offe = ((17 + jnp.arange(9, dtype=i32)) * VOCAB)[None, :]
    idx_n = jnp.concatenate([
        node_cat.astype(i32) + offn,
        lookahead_cat.astype(i32) + offl,
        package_postal.astype(i32) + 16 * VOCAB,
    ], axis=1).reshape(-1)
    idx_e = (edge_cat.astype(i32) + offe).reshape(-1)
    idx_all = jnp.concatenate([idx_n, idx_e])
    total = idx_all.shape[0]
    n_chunks = (total + 1023) // 1024
    pad = n_chunks * 1024 - total
    idx_all = jnp.concatenate([idx_all, jnp.zeros((pad,), i32)])
    idx2d = idx_all.reshape(n_chunks * 8, 128)

    gathered = _sc_embed_gather(stacked, idx2d, n_chunks)
    node_emb = gathered[:180000].reshape(N, 18 * EMBED)
    edge_emb = gathered[180000:180000 + 9 * E].reshape(E, 9 * EMBED)

    x_in = jnp.concatenate([x_cont, node_emb], axis=1)
    x = _tc_matmul_bias(x_in, W_node, b_node[None, :], 1024)
    ef_in = jnp.concatenate([edge_cont, edge_emb], axis=1)
    e_feat = _tc_matmul_bias(ef_in, W_edge, b_edge[None, :], 2048)

    src2d = edge_index[0].astype(i32).reshape(E // 128, 128)
    dst2d = edge_index[1].astype(i32).reshape(E // 128, 128)

    sel = jnp.array(np.repeat(np.eye(8, dtype=np.float32), DH, axis=0))
    z16 = jnp.zeros((N, 16), f32)
    z128 = jnp.zeros((N, HID), f32)

    for l in range(L):
        q, k, v, xr = _tc_proj4(x, Wq[l], Wk[l], Wv[l], Wskip[l],
                                bskip[l][None, :])
        qd, ks = _sc_edge_gather2(q, k, dst2d, src2d)
        logits, e_l, g16 = _tc_logits(qd, ks, e_feat, We[l], sel)
        g16 = g16.reshape(16)
        den = _sc_den_scatter(logits, g16, dst2d, z16)
        agg = _sc_agg_scatter(logits, g16, e_l, v, den[0], den[1],
                              dst2d, src2d, z128)
        wb = Wbeta[l]
        wba = wb[:HID] + wb[2 * HID:]
        wbx = wb[HID:2 * HID] - wb[2 * HID:]
        x = _tc_post(x, agg[0], agg[1], xr, wba, wbx,
                     ln1_g[l][None, :], ln1_b[l][None, :],
                     Wf1[l], bf1[l][None, :], Wf2[l], bf2[l][None, :],
                     ln2_g[l][None, :], ln2_b[l][None, :])
    return x
